# baseline (speedup 1.0000x reference)
import functools
import jax
import jax.numpy as jnp
from jax import lax
from jax.experimental import pallas as pl
from jax.experimental.pallas import tpu as pltpu
from jax.experimental.pallas import tpu_sc as plsc

_T0 = 1.0 / 256.0
_T1 = 2.0 / 256.0
_NW = 32
_SC_IMGS = 8
_RB = 64


def _sc_count_body(x_ref, out_ref, buf0, buf1, scr0, scr1, sem0, sem1):
    wid = lax.axis_index("c") * 16 + lax.axis_index("s")
    h, w = 512, 512
    blocks_per_img = 3 * (h // _RB)
    nchunks = _SC_IMGS * blocks_per_img // _NW

    def start(t, buf, sem):
        g = wid * nchunks + t
        img = g // blocks_per_img
        rem = g % blocks_per_img
        ch = rem // (h // _RB)
        rblk = rem % (h // _RB)
        pltpu.async_copy(x_ref.at[img, ch, pl.ds(rblk * _RB, _RB)], buf, sem)

    def wait(buf, sem):
        pltpu.make_async_copy(x_ref.at[0, 0, pl.ds(0, _RB)], buf, sem).wait()

    def chunk_counts(buf, accs):
        def row(r, accs):
            a0, a1 = accs
            for u in range(w // 16):
                v = buf[r, pl.ds(u * 16, 16)]
                a0 = a0 + jnp.where(v < _T0, 1.0, 0.0)
                a1 = a1 + jnp.where(v < _T1, 1.0, 0.0)
            return (a0, a1)

        return lax.fori_loop(0, _RB, row, accs)

    zero = jnp.zeros((16,), jnp.float32)
    accs = (zero, zero)
    start(0, buf0, sem0)
    for t in range(nchunks):
        buf, sem = (buf0, sem0) if t % 2 == 0 else (buf1, sem1)
        nbuf, nsem = (buf1, sem1) if t % 2 == 0 else (buf0, sem0)
        if t + 1 < nchunks:
            start(t + 1, nbuf, nsem)
        wait(buf, sem)
        accs = chunk_counts(buf, accs)

    scr0[...] = accs[0]
    scr1[...] = accs[1]
    pltpu.sync_copy(scr0, out_ref.at[0, wid])
    pltpu.sync_copy(scr1, out_ref.at[1, wid])


def _sc_counts(x):
    return pl.kernel(
        _sc_count_body,
        out_type=jax.ShapeDtypeStruct((2, _NW, 16), jnp.float32),
        mesh=plsc.VectorSubcoreMesh(core_axis_name="c", subcore_axis_name="s"),
        scratch_types=[
            pltpu.VMEM((_RB, 512), jnp.float32),
            pltpu.VMEM((_RB, 512), jnp.float32),
            pltpu.VMEM((16,), jnp.float32),
            pltpu.VMEM((16,), jnp.float32),
            pltpu.SemaphoreType.DMA,
            pltpu.SemaphoreType.DMA,
        ],
    )(x)


def _tc_count_body(x_ref, c_ref):
    i = pl.program_id(0)
    v = x_ref[...]
    p0 = jnp.sum((v < _T0).astype(jnp.int32))
    p1 = jnp.sum((v < _T1).astype(jnp.int32))

    @pl.when(i == 0)
    def _():
        c_ref[0] = p0
        c_ref[1] = p1

    @pl.when(i > 0)
    def _():
        c_ref[0] += p0
        c_ref[1] += p1


def _scale_body(total, ctc_ref, csc_ref, x_ref, o_ref):
    sc = jnp.sum(csc_ref[...], axis=(1, 2))
    c0 = ctc_ref[0].astype(jnp.float32) + sc[0]
    c1 = ctc_ref[1].astype(jnp.float32) + sc[1]
    o_ref[...] = x_ref[...] * ((c1 - c0) / (total - c0))


def kernel(x):
    n, c, h, w = x.shape
    total = float(x.size)

    csc = _sc_counts(x)

    ctc = pl.pallas_call(
        _tc_count_body,
        grid=(n - _SC_IMGS,),
        in_specs=[pl.BlockSpec((1, c, h, w), lambda i: (i + _SC_IMGS, 0, 0, 0))],
        out_specs=pl.BlockSpec(memory_space=pltpu.SMEM),
        out_shape=jax.ShapeDtypeStruct((2,), jnp.int32),
    )(x)

    blk = 2
    out = pl.pallas_call(
        lambda ct, cs, xr, o: _scale_body(total, ct, cs, xr, o),
        grid=(n // blk,),
        in_specs=[
            pl.BlockSpec(memory_space=pltpu.SMEM),
            pl.BlockSpec((2, _NW, 16), lambda i: (0, 0, 0)),
            pl.BlockSpec((blk, c, h, w), lambda i: (i, 0, 0, 0)),
        ],
        out_specs=pl.BlockSpec((blk, c, h, w), lambda i: (i, 0, 0, 0)),
        out_shape=jax.ShapeDtypeStruct(x.shape, jnp.float32),
    )(ctc, csc, x)

    return out


# speedup vs baseline: 1.1639x; 1.1639x over previous
import jax
import jax.numpy as jnp
from jax.experimental import pallas as pl
from jax.experimental.pallas import tpu as pltpu

_T0 = 1.0 / 256.0
_T1 = 2.0 / 256.0


def _fused_body(total, x_ref, o_ref, c_ref):
    p = pl.program_id(0)
    i = pl.program_id(1)
    v = x_ref[...]

    @pl.when(p == 0)
    def _():
        p0 = jnp.sum((v < _T0).astype(jnp.int32))
        p1 = jnp.sum((v < _T1).astype(jnp.int32))

        @pl.when(i == 0)
        def _():
            c_ref[0] = p0
            c_ref[1] = p1

        @pl.when(i > 0)
        def _():
            c_ref[0] += p0
            c_ref[1] += p1

    @pl.when(p == 1)
    def _():
        c0 = c_ref[0].astype(jnp.float32)
        c1 = c_ref[1].astype(jnp.float32)
        o_ref[...] = v * ((c1 - c0) / (total - c0))


def kernel(x):
    n, c, h, w = x.shape
    total = float(x.size)

    out = pl.pallas_call(
        lambda xr, o, cs: _fused_body(total, xr, o, cs),
        grid=(2, n // 2),
        in_specs=[pl.BlockSpec((2, c, h, w), lambda p, i: (i, 0, 0, 0))],
        out_specs=pl.BlockSpec((2, c, h, w), lambda p, i: (i * p, 0, 0, 0)),
        out_shape=jax.ShapeDtypeStruct(x.shape, jnp.float32),
        scratch_shapes=[pltpu.SMEM((2,), jnp.int32)],
        compiler_params=pltpu.CompilerParams(
            dimension_semantics=("arbitrary", "arbitrary"),
        ),
    )(x)

    return out
